# Initial kernel scaffold; baseline (speedup 1.0000x reference)
import jax
import jax.numpy as jnp
from jax.experimental import pallas as pl


def kernel(x, emb, w1, b1, w2, b2, w3, b3):
    raise NotImplementedError("write your pallas kernel here")


# speedup vs baseline: 1.5331x; 1.5331x over previous
import functools

import jax
import jax.numpy as jnp
from jax import lax
from jax.experimental import pallas as pl
from jax.experimental.pallas import tpu as pltpu
from jax.experimental.pallas import tpu_sc as plsc

_B = 4096
_H = 200
_D = 300
_DP = 384
_NC = 2
_NS = 16
_NW = _NC * _NS
_BPW = _B // _NW
_CH0 = 104
_CH1 = 96
_F1 = 150
_F2 = 150
_NV = _DP // 16


def _sc_pool_body(x_hbm, emb_hbm, out_hbm, idx_v, rows_v, pool_v, sem):
    wid = lax.axis_index("s") * _NC + lax.axis_index("c")
    base = wid * _BPW

    def elem_body(i, carry):
        b = base + i
        pltpu.sync_copy(x_hbm.at[pl.ds(pl.multiple_of(b * _H, 8), _H)], idx_v)
        cp0 = pltpu.async_copy(
            emb_hbm.at[idx_v.at[pl.ds(0, _CH0)]], rows_v.at[pl.ds(0, _CH0)], sem)
        cp1 = pltpu.async_copy(
            emb_hbm.at[idx_v.at[pl.ds(_CH0, _CH1)]],
            rows_v.at[pl.ds(_CH0, _CH1)], sem)
        cp0.wait()
        cp1.wait()
        zero = jnp.zeros((16,), jnp.float32)
        accs = (zero,) * _NV

        def row_body(r, a):
            a = list(a)
            for rr in (2 * r, 2 * r + 1):
                for j in range(_NV):
                    a[j] = a[j] + rows_v[rr, pl.ds(j * 16, 16)]
            return tuple(a)

        accs = lax.fori_loop(0, _H // 2, row_body, accs)
        for j in range(_NV):
            pool_v[pl.ds(j * 16, 16)] = accs[j]
        pltpu.sync_copy(pool_v, out_hbm.at[pl.ds(pl.multiple_of(b * _DP, 8), _DP)])
        return carry

    lax.fori_loop(0, _BPW, elem_body, 0)


_sc_pool = functools.partial(
    pl.kernel,
    mesh=plsc.VectorSubcoreMesh(core_axis_name="c", subcore_axis_name="s"),
    out_type=jax.ShapeDtypeStruct((_B * _DP,), jnp.float32),
    scratch_types=[
        pltpu.VMEM((_H,), jnp.int32),
        pltpu.VMEM((_H, _DP), jnp.float32),
        pltpu.VMEM((_DP,), jnp.float32),
        pltpu.SemaphoreType.DMA,
    ],
)(_sc_pool_body)


def _mlp_body(p_ref, w1_ref, b1_ref, w2_ref, b2_ref, w3_ref, b3_ref, o_ref):
    h = p_ref[...]
    h = jnp.dot(h, w1_ref[...], preferred_element_type=jnp.float32) + b1_ref[...]
    h = jnp.maximum(h, 0.0)
    h = jnp.dot(h, w2_ref[...], preferred_element_type=jnp.float32) + b2_ref[...]
    h = jnp.maximum(h, 0.0)
    o_ref[...] = (
        jnp.dot(h, w3_ref[...], preferred_element_type=jnp.float32) + b3_ref[...])


def kernel(x, emb, w1, b1, w2, b2, w3, b3):
    x = x.astype(jnp.int32)
    emb_p = jnp.pad(emb, ((0, 0), (0, _DP - _D)))
    pooled = _sc_pool(x.reshape(-1), emb_p).reshape(_B, _DP)
    w1p = jnp.zeros((_DP, _F1), jnp.float32).at[:_D].set(w1.T * (1.0 / _H))
    blk = 1024
    out = pl.pallas_call(
        _mlp_body,
        grid=(_B // blk,),
        in_specs=[
            pl.BlockSpec((blk, _DP), lambda i: (i, 0)),
            pl.BlockSpec((_DP, _F1), lambda i: (0, 0)),
            pl.BlockSpec((1, _F1), lambda i: (0, 0)),
            pl.BlockSpec((_F1, _F2), lambda i: (0, 0)),
            pl.BlockSpec((1, _F2), lambda i: (0, 0)),
            pl.BlockSpec((_F2, 1), lambda i: (0, 0)),
            pl.BlockSpec((1, 1), lambda i: (0, 0)),
        ],
        out_specs=pl.BlockSpec((blk, 1), lambda i: (i, 0)),
        out_shape=jax.ShapeDtypeStruct((_B, 1), jnp.float32),
    )(pooled, w1p, b1.reshape(1, _F1), w2.T, b2.reshape(1, _F2),
      w3.T, b3.reshape(1, 1))
    return out


# speedup vs baseline: 3.0383x; 1.9818x over previous
import functools

import jax
import jax.numpy as jnp
from jax import lax
from jax.experimental import pallas as pl
from jax.experimental.pallas import tpu as pltpu
from jax.experimental.pallas import tpu_sc as plsc

_B = 4096
_H = 200
_D = 300
_DP = 384
_NC = 2
_NS = 16
_NW = _NC * _NS
_BPW = _B // _NW
_CH0 = 104
_CH1 = 96
_F1 = 150
_F2 = 150
_NV = _DP // 16


def _sc_pool_body(x_hbm, emb_hbm, out_hbm, idx_v, rows_v, pool_v, sem0, sem1):
    wid = lax.axis_index("s") * _NC + lax.axis_index("c")
    base = wid * _BPW
    pltpu.sync_copy(
        x_hbm.at[pl.ds(pl.multiple_of(base * _H, 8), _BPW * _H)], idx_v)

    def idx_slice(i, c):
        off = pl.multiple_of(i * _H + c * _CH0, 8)
        return idx_v.at[pl.ds(off, _CH1 if c else _CH0)]

    def gather(i, c):
        dst = rows_v.at[c, pl.ds(0, _CH1 if c else _CH0)]
        sem = sem1 if c else sem0
        return pltpu.make_async_copy(emb_hbm.at[idx_slice(i, c)], dst, sem)

    def accum(accs, slot, nrows):
        def row_body(r, a):
            a = list(a)
            for rr in (2 * r, 2 * r + 1):
                for j in range(_NV):
                    a[j] = a[j] + rows_v[slot, rr, pl.ds(j * 16, 16)]
            return tuple(a)
        return lax.fori_loop(0, nrows // 2, row_body, accs)

    gather(0, 0).start()

    def elem_body(i, carry):
        gather(i, 1).start()
        gather(i, 0).wait()
        accs = accum((jnp.zeros((16,), jnp.float32),) * _NV, 0, _CH0)

        @pl.when(i + 1 < _BPW)
        def _():
            gather(i + 1, 0).start()

        gather(i, 1).wait()
        accs = accum(accs, 1, _CH1)
        for j in range(_NV):
            pool_v[pl.ds(j * 16, 16)] = accs[j]
        b = base + i
        pltpu.sync_copy(
            pool_v, out_hbm.at[pl.ds(pl.multiple_of(b * _DP, 8), _DP)])
        return carry

    lax.fori_loop(0, _BPW, elem_body, 0)


_sc_pool = functools.partial(
    pl.kernel,
    mesh=plsc.VectorSubcoreMesh(core_axis_name="c", subcore_axis_name="s"),
    out_type=jax.ShapeDtypeStruct((_B * _DP,), jnp.float32),
    scratch_types=[
        pltpu.VMEM((_BPW * _H,), jnp.int32),
        pltpu.VMEM((2, _CH0, _DP), jnp.float32),
        pltpu.VMEM((_DP,), jnp.float32),
        pltpu.SemaphoreType.DMA,
        pltpu.SemaphoreType.DMA,
    ],
)(_sc_pool_body)


def _pad_body(e_ref, o_ref):
    blk = e_ref.shape[0]
    o_ref[...] = jnp.concatenate(
        [e_ref[...], jnp.zeros((blk, _DP - _D), jnp.float32)], axis=1)


def _mlp_body(p_ref, w1_ref, b1_ref, w2_ref, b2_ref, w3_ref, b3_ref, o_ref):
    h = p_ref[...]
    h = jnp.dot(h, w1_ref[...], preferred_element_type=jnp.float32) + b1_ref[...]
    h = jnp.maximum(h, 0.0)
    h = jnp.dot(h, w2_ref[...], preferred_element_type=jnp.float32) + b2_ref[...]
    h = jnp.maximum(h, 0.0)
    o_ref[...] = (
        jnp.dot(h, w3_ref[...], preferred_element_type=jnp.float32) + b3_ref[...])


def kernel(x, emb, w1, b1, w2, b2, w3, b3):
    x = x.astype(jnp.int32)
    vblk = 2000
    emb_p = pl.pallas_call(
        _pad_body,
        grid=(emb.shape[0] // vblk,),
        in_specs=[pl.BlockSpec((vblk, _D), lambda i: (i, 0))],
        out_specs=pl.BlockSpec((vblk, _DP), lambda i: (i, 0)),
        out_shape=jax.ShapeDtypeStruct((emb.shape[0], _DP), jnp.float32),
    )(emb)
    pooled = _sc_pool(x.reshape(-1), emb_p).reshape(_B, _DP)
    w1p = jnp.zeros((_DP, _F1), jnp.float32).at[:_D].set(w1.T * (1.0 / _H))
    blk = 1024
    out = pl.pallas_call(
        _mlp_body,
        grid=(_B // blk,),
        in_specs=[
            pl.BlockSpec((blk, _DP), lambda i: (i, 0)),
            pl.BlockSpec((_DP, _F1), lambda i: (0, 0)),
            pl.BlockSpec((1, _F1), lambda i: (0, 0)),
            pl.BlockSpec((_F1, _F2), lambda i: (0, 0)),
            pl.BlockSpec((1, _F2), lambda i: (0, 0)),
            pl.BlockSpec((_F2, 1), lambda i: (0, 0)),
            pl.BlockSpec((1, 1), lambda i: (0, 0)),
        ],
        out_specs=pl.BlockSpec((blk, 1), lambda i: (i, 0)),
        out_shape=jax.ShapeDtypeStruct((_B, 1), jnp.float32),
    )(pooled, w1p, b1.reshape(1, _F1), w2.T, b2.reshape(1, _F2),
      w3.T, b3.reshape(1, 1))
    return out
